# Initial kernel scaffold; baseline (speedup 1.0000x reference)
import jax
import jax.numpy as jnp
from jax.experimental import pallas as pl


def kernel(x, adj, W1, b1, W2, b2):
    raise NotImplementedError("write your pallas kernel here")


# speedup vs baseline: 20.2706x; 20.2706x over previous
import jax
import jax.numpy as jnp
from jax.experimental import pallas as pl


def _s1_kernel(x_ref, w1_ref, o_ref):
    o_ref[...] = jnp.dot(x_ref[...], w1_ref[...],
                         preferred_element_type=jnp.float32)


def _layer1_kernel(adj_ref, s1_ref, b1_ref, w2_ref, o_ref):
    h = jnp.dot(adj_ref[...], s1_ref[...],
                preferred_element_type=jnp.float32) + b1_ref[...]
    h = jnp.maximum(h, 0.0)
    o_ref[...] = jnp.dot(h, w2_ref[...], preferred_element_type=jnp.float32)


def _layer2_kernel(adj_ref, s2_ref, b2_ref, o_ref):
    o_ref[...] = jnp.dot(adj_ref[...], s2_ref[...],
                         preferred_element_type=jnp.float32) + b2_ref[...]


def _decode_kernel(zi_ref, zjt_ref, o_ref):
    zz = jnp.dot(zi_ref[...], zjt_ref[...],
                 preferred_element_type=jnp.float32)
    o_ref[...] = jax.nn.sigmoid(zz)


def kernel(x, adj, W1, b1, W2, b2):
    N, F = x.shape
    H = W1.shape[1]
    C = W2.shape[1]
    b1r = b1.reshape(1, H)
    b2r = b2.reshape(1, C)

    s1 = pl.pallas_call(
        _s1_kernel,
        out_shape=jax.ShapeDtypeStruct((N, H), jnp.float32),
    )(x, W1)

    BI = 400
    G = N // BI
    s2 = pl.pallas_call(
        _layer1_kernel,
        grid=(G,),
        in_specs=[
            pl.BlockSpec((BI, N), lambda i: (i, 0)),
            pl.BlockSpec((N, H), lambda i: (0, 0)),
            pl.BlockSpec((1, H), lambda i: (0, 0)),
            pl.BlockSpec((H, C), lambda i: (0, 0)),
        ],
        out_specs=pl.BlockSpec((BI, C), lambda i: (i, 0)),
        out_shape=jax.ShapeDtypeStruct((N, C), jnp.float32),
    )(adj, s1, b1r, W2)

    z = pl.pallas_call(
        _layer2_kernel,
        grid=(G,),
        in_specs=[
            pl.BlockSpec((BI, N), lambda i: (i, 0)),
            pl.BlockSpec((N, C), lambda i: (0, 0)),
            pl.BlockSpec((1, C), lambda i: (0, 0)),
        ],
        out_specs=pl.BlockSpec((BI, C), lambda i: (i, 0)),
        out_shape=jax.ShapeDtypeStruct((N, C), jnp.float32),
    )(adj, s2, b2r)

    zt = z.T

    BD = 400
    GD = N // BD
    a_pred = pl.pallas_call(
        _decode_kernel,
        grid=(GD,),
        in_specs=[
            pl.BlockSpec((BD, C), lambda i: (i, 0)),
            pl.BlockSpec((C, N), lambda i: (0, 0)),
        ],
        out_specs=pl.BlockSpec((BD, N), lambda i: (i, 0)),
        out_shape=jax.ShapeDtypeStruct((N, N), jnp.float32),
    )(z, zt)
    return a_pred
